# Initial kernel scaffold; baseline (speedup 1.0000x reference)
import jax
import jax.numpy as jnp
from jax.experimental import pallas as pl


def kernel(x, batch, W, b):
    raise NotImplementedError("write your pallas kernel here")


# speedup vs baseline: 4.7846x; 4.7846x over previous
import jax
import jax.numpy as jnp
from jax import lax
from jax.experimental import pallas as pl
from jax.experimental.pallas import tpu as pltpu
from jax.experimental.pallas import tpu_sc as plsc

N = 100000
D = 128
G = 512
T = 12

NC = 2
NS = 16
NW = NC * NS
L = 16

R = 128
FULL = N // R
TAIL = N - FULL * R
KMAX = (FULL + NW - 1) // NW
BSTEPS = 17


def _zero_2d(ref, rows, cols):
    z = jnp.zeros((L,), jnp.float32)

    def body(i, carry):
        for j in range(cols // L):
            ref[i, pl.ds(j * L, L)] = z
        return carry

    lax.fori_loop(0, rows, body, 0)


def _lower_bound(batch_v, targets):
    lo0 = jnp.zeros((L,), jnp.int32)
    hi0 = jnp.full((L,), N, jnp.int32)

    def step(_, carry):
        lo, hi = carry
        mid = lax.div(lo + hi, jnp.int32(2))
        mid_c = jnp.minimum(mid, jnp.full((L,), N - 1, jnp.int32))
        vals = plsc.load_gather(batch_v, [mid_c])
        pred = vals >= targets
        active = lo < hi
        hi = jnp.where(jnp.logical_and(pred, active), mid_c, hi)
        lo = jnp.where(jnp.logical_and(jnp.logical_not(pred), active),
                       mid_c + 1, lo)
        return lo, hi

    lo, _ = lax.fori_loop(0, BSTEPS, step, (lo0, hi0))
    return lo


def _sc_segment_sums(x, batch):
    mesh = plsc.VectorSubcoreMesh(core_axis_name="c", subcore_axis_name="s")

    def body(x_hbm, batch_hbm, parts_hbm, cnts_hbm,
             xbuf, idx_v, idx_t, zrow, batch_v, cnt_v, acc_sh):
        cid = lax.axis_index("c")
        sid = lax.axis_index("s")
        wid = sid * NC + cid

        rows_per_tile = G // NS
        _zero_2d(zrow, rows_per_tile, D)
        pltpu.sync_copy(zrow, acc_sh.at[pl.ds(sid * rows_per_tile, rows_per_tile)])
        plsc.subcore_barrier()

        @pl.when(wid == 0)
        def _():
            pltpu.sync_copy(batch_hbm, batch_v)
            lane = lax.broadcasted_iota(jnp.int32, (L,), 0)

            def cnt_body(vb, carry):
                g0 = vb * L
                lb_lo = _lower_bound(batch_v, g0 + lane)
                lb_hi = _lower_bound(batch_v, g0 + 1 + lane)
                cnt_v[pl.ds(g0, L)] = (lb_hi - lb_lo).astype(jnp.float32)
                return carry

            lax.fori_loop(0, G // L, cnt_body, 0)
            pltpu.sync_copy(cnt_v, cnts_hbm)

        def chunk_body(k, carry):
            c = wid + k * NW

            @pl.when(c < FULL)
            def _():
                base = c * R
                pltpu.sync_copy(batch_hbm.at[pl.ds(base, R)], idx_v)
                pltpu.sync_copy(x_hbm.at[pl.ds(base, R)], xbuf)
                pltpu.sync_copy(xbuf, acc_sh.at[idx_v], add=True)

            return carry

        lax.fori_loop(0, KMAX, chunk_body, 0)

        @pl.when(wid == NW - 1)
        def _():
            base = FULL * R
            pltpu.sync_copy(batch_hbm.at[pl.ds(base, TAIL)], idx_t)
            pltpu.sync_copy(x_hbm.at[pl.ds(base, TAIL)], xbuf.at[pl.ds(0, TAIL)])
            pltpu.sync_copy(xbuf.at[pl.ds(0, TAIL)], acc_sh.at[idx_t], add=True)

        plsc.subcore_barrier()

        lo = sid * rows_per_tile
        pltpu.sync_copy(acc_sh.at[pl.ds(lo, rows_per_tile)],
                        parts_hbm.at[cid, pl.ds(lo, rows_per_tile)])

    return pl.kernel(
        body,
        out_type=(
            jax.ShapeDtypeStruct((NC, G, D), jnp.float32),
            jax.ShapeDtypeStruct((G,), jnp.float32),
        ),
        mesh=mesh,
        scratch_types=[
            pltpu.VMEM((R, D), jnp.float32),
            pltpu.VMEM((R,), jnp.int32),
            pltpu.VMEM((TAIL,), jnp.int32),
            pltpu.VMEM((G // NS, D), jnp.float32),
            pltpu.VMEM((N,), jnp.int32),
            pltpu.VMEM((G,), jnp.float32),
            pltpu.VMEM_SHARED((G, D), jnp.float32),
        ],
        compiler_params=pltpu.CompilerParams(needs_layout_passes=False),
    )(x, batch)


def _head_body(parts_ref, cnts_ref, w_ref, b_ref, o_ref):
    sums = parts_ref[0] + parts_ref[1]
    rep = sums / jnp.maximum(cnts_ref[...], 1.0)
    o_ref[...] = (
        jnp.dot(rep, w_ref[...], preferred_element_type=jnp.float32)
        + b_ref[...]
    )


def kernel(x, batch, W, b):
    parts, cnts = _sc_segment_sums(x, batch.astype(jnp.int32))
    out = pl.pallas_call(
        _head_body,
        out_shape=jax.ShapeDtypeStruct((G, T), jnp.float32),
    )(parts, cnts.reshape(G, 1), W, b.reshape(1, T))
    return out


# speedup vs baseline: 6.3105x; 1.3189x over previous
import jax
import jax.numpy as jnp
from jax import lax
from jax.experimental import pallas as pl
from jax.experimental.pallas import tpu as pltpu
from jax.experimental.pallas import tpu_sc as plsc

N = 100000
D = 128
G = 512
T = 12

NC = 2
NS = 16
NW = NC * NS
L = 16

CH = 256
NCH = (N // CH)
TAIL_BASE = NCH * CH
TAIL_ROWS = N - TAIL_BASE
NSTREAM = NW - 2
KPW = NCH // NSTREAM
BSTEPS = 17
NPACK = N // 2


def _zero_rows(ref, row0, rows):
    z = jnp.zeros((L,), jnp.float32)

    def body(i, carry):
        for j in range(D // L):
            ref[i, pl.ds(j * L, L)] = z
        return carry

    lax.fori_loop(row0, row0 + rows, body, 0)


def _lb_packed(pv, targets):
    lo = jnp.zeros((L,), jnp.int32)
    hi = jnp.full((L,), N, jnp.int32)
    nm1 = jnp.full((L,), N - 1, jnp.int32)
    one = jnp.full((L,), 1, jnp.int32)
    for _ in range(BSTEPS):
        mid = lax.shift_right_logical(lo + hi, one)
        midc = jnp.minimum(mid, nm1)
        w = plsc.load_gather(pv, [lax.shift_right_logical(midc, one)])
        sh = lax.shift_left(jnp.bitwise_and(midc, one), jnp.full((L,), 4, jnp.int32))
        val = jnp.bitwise_and(lax.shift_right_logical(w, sh),
                              jnp.full((L,), 0xFFFF, jnp.int32))
        pred = val >= targets
        act = lo < hi
        hi = jnp.where(jnp.logical_and(pred, act), midc, hi)
        lo = jnp.where(jnp.logical_and(jnp.logical_not(pred), act),
                       midc + 1, lo)
    return lo


def _sc_segment_sums(x, batch2, btail, packed):
    mesh = plsc.VectorSubcoreMesh(core_axis_name="c", subcore_axis_name="s")

    def body(x_hbm, batch2_hbm, btail_hbm, packed_hbm, parts_hbm, cnts_hbm,
             xbuf0, xbuf1, idxb0, idxb1, packed_v, cnt_v,
             semx0, semx1, semi0, semi1, acc_sh):
        cid = lax.axis_index("c")
        sid = lax.axis_index("s")
        wid = sid * NC + cid
        xbuf = (xbuf0, xbuf1)
        idxb = (idxb0, idxb1)
        semx = (semx0, semx1)
        semi = (semi0, semi1)

        rows_per_tile = G // NS
        _zero_rows(xbuf0, 0, rows_per_tile)
        pltpu.sync_copy(xbuf0.at[pl.ds(0, rows_per_tile)],
                        acc_sh.at[pl.ds(sid * rows_per_tile, rows_per_tile)])
        plsc.subcore_barrier()

        @pl.when(wid == 1)
        def _():
            _zero_rows(xbuf0, TAIL_ROWS, CH - TAIL_ROWS)
            pltpu.sync_copy(x_hbm.at[pl.ds(TAIL_BASE, TAIL_ROWS)],
                            xbuf0.at[pl.ds(0, TAIL_ROWS)])
            pltpu.sync_copy(btail_hbm, idxb0)
            for h in range(2):
                pltpu.sync_copy(xbuf0.at[pl.ds(h * 128, 128)],
                                acc_sh.at[idxb0.at[h]], add=True)

        @pl.when(wid < 2)
        def _():
            pltpu.sync_copy(packed_hbm, packed_v)
            lane = lax.broadcasted_iota(jnp.int32, (L,), 0)
            half = wid * (G // 2)

            def cnt_body(t, carry):
                g0 = half + t * L
                lb_lo = _lb_packed(packed_v, g0 + lane)
                lb_hi = _lb_packed(packed_v, g0 + 1 + lane)
                cnt_v[pl.ds(g0, L)] = (lb_hi - lb_lo).astype(jnp.float32)
                return carry

            lax.fori_loop(0, G // 2 // L, cnt_body, 0)
            pltpu.sync_copy(cnt_v.at[pl.ds(half, G // 2)],
                            cnts_hbm.at[pl.ds(half, G // 2)])

        @pl.when(wid >= 2)
        def _():
            j = wid - 2

            def issue(k, b):
                c = j + NSTREAM * k
                di = pltpu.async_copy(batch2_hbm.at[c], idxb[b], semi[b])
                dx = pltpu.async_copy(x_hbm.at[pl.ds(c * CH, CH)], xbuf[b],
                                      semx[b])
                return di, dx

            descs = {0: issue(0, 0), 1: issue(1, 1)}
            for k in range(KPW):
                b = k & 1
                di, dx = descs.pop(k)
                di.wait()
                dx.wait()
                for h in range(2):
                    pltpu.sync_copy(xbuf[b].at[pl.ds(h * 128, 128)],
                                    acc_sh.at[idxb[b].at[h]], add=True)
                if k + 2 < KPW:
                    descs[k + 2] = issue(k + 2, b)

        plsc.subcore_barrier()

        lo = sid * rows_per_tile
        pltpu.sync_copy(acc_sh.at[pl.ds(lo, rows_per_tile)],
                        parts_hbm.at[cid, pl.ds(lo, rows_per_tile)])

    return pl.kernel(
        body,
        out_type=(
            jax.ShapeDtypeStruct((NC, G, D), jnp.float32),
            jax.ShapeDtypeStruct((G,), jnp.float32),
        ),
        mesh=mesh,
        scratch_types=[
            pltpu.VMEM((CH, D), jnp.float32),
            pltpu.VMEM((CH, D), jnp.float32),
            pltpu.VMEM((2, 128), jnp.int32),
            pltpu.VMEM((2, 128), jnp.int32),
            pltpu.VMEM((NPACK,), jnp.int32),
            pltpu.VMEM((G,), jnp.float32),
            pltpu.SemaphoreType.DMA,
            pltpu.SemaphoreType.DMA,
            pltpu.SemaphoreType.DMA,
            pltpu.SemaphoreType.DMA,
            pltpu.VMEM_SHARED((G, D), jnp.float32),
        ],
        compiler_params=pltpu.CompilerParams(needs_layout_passes=False),
    )(x, batch2, btail, packed)


def _head_body(parts_ref, cnts_ref, w_ref, b_ref, o_ref):
    sums = parts_ref[0] + parts_ref[1]
    rep = sums / jnp.maximum(cnts_ref[...], 1.0)
    o_ref[...] = (
        jnp.dot(rep, w_ref[...], preferred_element_type=jnp.float32)
        + b_ref[...]
    )


def kernel(x, batch, W, b):
    bi = batch.astype(jnp.int32)
    batch2 = bi[:TAIL_BASE].reshape(NCH, 2, 128)
    btail = jnp.concatenate(
        [bi[TAIL_BASE:], jnp.zeros((CH - TAIL_ROWS,), jnp.int32)]
    ).reshape(2, 128)
    packed = jnp.bitwise_or(bi[0::2], jnp.left_shift(bi[1::2], 16))
    parts, cnts = _sc_segment_sums(x, batch2, btail, packed)
    out = pl.pallas_call(
        _head_body,
        out_shape=jax.ShapeDtypeStruct((G, T), jnp.float32),
    )(parts, cnts.reshape(G, 1), W, b.reshape(1, T))
    return out


# speedup vs baseline: 8.6461x; 1.3701x over previous
import jax
import jax.numpy as jnp
from jax import lax
from jax.experimental import pallas as pl
from jax.experimental.pallas import tpu as pltpu
from jax.experimental.pallas import tpu_sc as plsc

N = 100000
D = 128
G = 512
T = 12

NC = 2
NS = 16
NW = NC * NS
L = 16

CH = 256
NCH = (N // CH)
TAIL_BASE = NCH * CH
TAIL_ROWS = N - TAIL_BASE
NSTREAM = NW - 2
KPW = NCH // NSTREAM
BSTEPS = 17
NPACK = N // 2


def _zero_rows(ref, row0, rows):
    z = jnp.zeros((L,), jnp.float32)

    def body(i, carry):
        for j in range(D // L):
            ref[i, pl.ds(j * L, L)] = z
        return carry

    lax.fori_loop(row0, row0 + rows, body, 0)


def _lb_packed(pv, targets):
    half = jnp.full((L,), NPACK, jnp.int32)
    lo = jnp.zeros((L,), jnp.int32)
    hi = jnp.full((L,), N, jnp.int32)
    nm1 = jnp.full((L,), N - 1, jnp.int32)
    one = jnp.full((L,), 1, jnp.int32)
    for _ in range(BSTEPS):
        mid = lax.shift_right_logical(lo + hi, one)
        midc = jnp.minimum(mid, nm1)
        in_lo = midc < half
        word = jnp.where(in_lo, midc, midc - NPACK)
        w = plsc.load_gather(pv, [word])
        sh = jnp.where(in_lo, jnp.zeros((L,), jnp.int32),
                       jnp.full((L,), 16, jnp.int32))
        val = jnp.bitwise_and(lax.shift_right_logical(w, sh),
                              jnp.full((L,), 0xFFFF, jnp.int32))
        pred = val >= targets
        act = lo < hi
        hi = jnp.where(jnp.logical_and(pred, act), midc, hi)
        lo = jnp.where(jnp.logical_and(jnp.logical_not(pred), act),
                       midc + 1, lo)
    return lo


def _sc_segment_sums(x, batch, packed):
    mesh = plsc.VectorSubcoreMesh(core_axis_name="c", subcore_axis_name="s")

    def body(x_hbm, batch_hbm, packed_hbm, parts_hbm, cnts_hbm,
             xbuf0, xbuf1, idxb0, idxb1, packed_v, cnt_v,
             semx0, semx1, semi0, semi1, acc_sh):
        cid = lax.axis_index("c")
        sid = lax.axis_index("s")
        wid = sid * NC + cid
        xbuf = (xbuf0, xbuf1)
        idxb = (idxb0, idxb1)
        semx = (semx0, semx1)
        semi = (semi0, semi1)

        rows_per_tile = G // NS
        _zero_rows(xbuf0, 0, rows_per_tile)
        pltpu.sync_copy(xbuf0.at[pl.ds(0, rows_per_tile)],
                        acc_sh.at[pl.ds(sid * rows_per_tile, rows_per_tile)])
        plsc.subcore_barrier()

        @pl.when(wid == 1)
        def _():
            _zero_rows(xbuf0, TAIL_ROWS, CH - TAIL_ROWS)
            zi = jnp.zeros((L,), jnp.int32)
            for j in range((TAIL_ROWS - 128) // L, 128 // L):
                idxb0[1, pl.ds(j * L, L)] = zi
            pltpu.sync_copy(x_hbm.at[pl.ds(TAIL_BASE, TAIL_ROWS)],
                            xbuf0.at[pl.ds(0, TAIL_ROWS)])
            pltpu.sync_copy(batch_hbm.at[pl.ds(TAIL_BASE, 128)], idxb0.at[0])
            pltpu.sync_copy(batch_hbm.at[pl.ds(TAIL_BASE + 128, TAIL_ROWS - 128)],
                            idxb0.at[1, pl.ds(0, TAIL_ROWS - 128)])
            for h in range(2):
                pltpu.sync_copy(xbuf0.at[pl.ds(h * 128, 128)],
                                acc_sh.at[idxb0.at[h]], add=True)

        @pl.when(wid < 2)
        def _():
            pltpu.sync_copy(packed_hbm, packed_v)
            lane = lax.broadcasted_iota(jnp.int32, (L,), 0)
            half = wid * (G // 2)

            def cnt_body(t, carry):
                g0 = half + t * L
                lb_lo = _lb_packed(packed_v, g0 + lane)
                lb_hi = _lb_packed(packed_v, g0 + 1 + lane)
                cnt_v[pl.ds(g0, L)] = (lb_hi - lb_lo).astype(jnp.float32)
                return carry

            lax.fori_loop(0, G // 2 // L, cnt_body, 0)
            pltpu.sync_copy(cnt_v.at[pl.ds(half, G // 2)],
                            cnts_hbm.at[pl.ds(half, G // 2)])

        @pl.when(wid >= 2)
        def _():
            j = wid - 2

            def issue(k, b):
                c = j + NSTREAM * k
                di0 = pltpu.async_copy(batch_hbm.at[pl.ds(c * CH, 128)],
                                       idxb[b].at[0], semi[b])
                di1 = pltpu.async_copy(batch_hbm.at[pl.ds(c * CH + 128, 128)],
                                       idxb[b].at[1], semi[b])
                dx = pltpu.async_copy(x_hbm.at[pl.ds(c * CH, CH)], xbuf[b],
                                      semx[b])
                return di0, di1, dx

            descs = {0: issue(0, 0), 1: issue(1, 1)}
            for k in range(KPW):
                b = k & 1
                di0, di1, dx = descs.pop(k)
                di0.wait()
                di1.wait()
                dx.wait()
                for h in range(2):
                    pltpu.sync_copy(xbuf[b].at[pl.ds(h * 128, 128)],
                                    acc_sh.at[idxb[b].at[h]], add=True)
                if k + 2 < KPW:
                    descs[k + 2] = issue(k + 2, b)

        plsc.subcore_barrier()

        lo = sid * rows_per_tile
        pltpu.sync_copy(acc_sh.at[pl.ds(lo, rows_per_tile)],
                        parts_hbm.at[cid, pl.ds(lo, rows_per_tile)])

    return pl.kernel(
        body,
        out_type=(
            jax.ShapeDtypeStruct((NC, G, D), jnp.float32),
            jax.ShapeDtypeStruct((G,), jnp.float32),
        ),
        mesh=mesh,
        scratch_types=[
            pltpu.VMEM((CH, D), jnp.float32),
            pltpu.VMEM((CH, D), jnp.float32),
            pltpu.VMEM((2, 128), jnp.int32),
            pltpu.VMEM((2, 128), jnp.int32),
            pltpu.VMEM((NPACK,), jnp.int32),
            pltpu.VMEM((G,), jnp.float32),
            pltpu.SemaphoreType.DMA,
            pltpu.SemaphoreType.DMA,
            pltpu.SemaphoreType.DMA,
            pltpu.SemaphoreType.DMA,
            pltpu.VMEM_SHARED((G, D), jnp.float32),
        ],
        compiler_params=pltpu.CompilerParams(needs_layout_passes=False),
    )(x, batch, packed)


def _head_body(parts_ref, cnts_ref, w_ref, b_ref, o_ref):
    sums = parts_ref[0] + parts_ref[1]
    rep = sums / jnp.maximum(cnts_ref[...], 1.0)
    o_ref[...] = (
        jnp.dot(rep, w_ref[...], preferred_element_type=jnp.float32)
        + b_ref[...]
    )


def kernel(x, batch, W, b):
    bi = batch.astype(jnp.int32)
    packed = jnp.bitwise_or(bi[:NPACK], jnp.left_shift(bi[NPACK:], 16))
    parts, cnts = _sc_segment_sums(x, bi, packed)
    out = pl.pallas_call(
        _head_body,
        out_shape=jax.ShapeDtypeStruct((G, T), jnp.float32),
    )(parts, cnts.reshape(G, 1), W, b.reshape(1, T))
    return out
